# baseline (speedup 1.0000x reference)
import jax
import jax.numpy as jnp
from jax import lax
from jax.experimental import pallas as pl
from jax.experimental.pallas import tpu as pltpu
from jax.experimental.pallas import tpu_sc as plsc

_N = 10000
_D = 256
_NET = 4
_STEPS = 4
_BN = 2000
_NB = _N // _BN
_NPAD = 10112
_CH = 64
_NCH = 160
_EPW = _NCH * _CH
_NSUB = 16
_EPAD = _NSUB * _EPW
_TROWS = 2 * _NET * _N
_RPS = _NPAD // _NSUB
_PREC = lax.Precision.DEFAULT


def _init_body(x_ref, w_ref, b_ref, o_ref):
    o_ref[...] = lax.dot_general(
        x_ref[...], w_ref[...], (((1,), (1,)), ((), ())),
        precision=_PREC, preferred_element_type=jnp.float32) + b_ref[...]


def _proj_body(h_ref, w_ref, b_ref, o_ref):
    e = pl.program_id(1)
    hf = pl.program_id(2)
    b8 = b_ref[...]
    rowid = lax.broadcasted_iota(jnp.int32, (2 * _NET, 128), 0)
    b = jnp.sum(jnp.where(rowid == e * 2 + hf, b8, 0.0), axis=0, keepdims=True)
    o_ref[...] = lax.dot_general(
        h_ref[...], w_ref[0], (((1,), (1,)), ((), ())),
        precision=_PREC, preferred_element_type=jnp.float32) + b


def _gru_body(h_ref, alo_ref, ahi_ref, wih_ref, whh_ref, bih_ref, bhh_ref,
              o_ref):
    h = h_ref[...]
    a = jnp.concatenate([alo_ref[...], ahi_ref[...]], axis=1)
    gi = lax.dot_general(a, wih_ref[...], (((1,), (1,)), ((), ())),
                         precision=_PREC,
                         preferred_element_type=jnp.float32) + bih_ref[...]
    gh = lax.dot_general(h, whh_ref[...], (((1,), (1,)), ((), ())),
                         precision=_PREC,
                         preferred_element_type=jnp.float32) + bhh_ref[...]
    r = jax.nn.sigmoid(gi[:, :_D] + gh[:, :_D])
    z = jax.nn.sigmoid(gi[:, _D:2 * _D] + gh[:, _D:2 * _D])
    n = jnp.tanh(gi[:, 2 * _D:] + r * gh[:, 2 * _D:])
    o_ref[...] = (1.0 - z) * n + z * h


def _pool_body(h_ref, gw_ref, cw_ref, cb_ref, o_ref, m_s, s_s, v_s):
    i = pl.program_id(0)
    h = h_ref[...]
    g = lax.dot_general(h, gw_ref[...], (((1,), (1,)), ((), ())),
                        precision=_PREC, preferred_element_type=jnp.float32)
    first = i == 0
    m_old = jnp.where(first, -jnp.inf, m_s[0, 0])
    s_old = jnp.where(first, 0.0, s_s[0, 0])
    v_old = jnp.where(first, jnp.zeros((1, _D), jnp.float32), v_s[...])
    bm = jnp.max(g)
    m_new = jnp.maximum(m_old, bm)
    corr = jnp.exp(m_old - m_new)
    p = jnp.exp(g - m_new)
    s_new = s_old * corr + jnp.sum(p)
    pv = lax.dot_general(p, h, (((0,), (0,)), ((), ())),
                         precision=_PREC,
                         preferred_element_type=jnp.float32)
    v_new = v_old * corr + pv
    m_s[0, 0] = m_new
    s_s[0, 0] = s_new
    v_s[...] = v_new

    @pl.when(i == _NB - 1)
    def _():
        readout = v_new / s_new
        o_ref[...] = lax.dot_general(
            readout, cw_ref[...], (((1,), (1,)), ((), ())),
            precision=_PREC, preferred_element_type=jnp.float32) + cb_ref[...]


def _etype_bias(b_ref, e, hf):
    rowid = lax.broadcasted_iota(jnp.int32, (2 * _NET, 128), 0)
    return jnp.sum(jnp.where(rowid == e * 2 + hf, b_ref[...], 0.0), axis=0,
                   keepdims=True)


def _gru_math(h, a, wih_ref, whh_ref, bih_ref, bhh_ref):
    gi = lax.dot_general(a, wih_ref[...], (((1,), (1,)), ((), ())),
                         precision=_PREC,
                         preferred_element_type=jnp.float32) + bih_ref[...]
    gh = lax.dot_general(h, whh_ref[...], (((1,), (1,)), ((), ())),
                         precision=_PREC,
                         preferred_element_type=jnp.float32) + bhh_ref[...]
    r = jax.nn.sigmoid(gi[:, :_D] + gh[:, :_D])
    z = jax.nn.sigmoid(gi[:, _D:2 * _D] + gh[:, _D:2 * _D])
    n = jnp.tanh(gi[:, 2 * _D:] + r * gh[:, 2 * _D:])
    return (1.0 - z) * n + z * h


def _initproj_body(x_ref, wl_ref, bl_ref, wet_ref, bet_ref, tab_ref, h_ref,
                   hs_v):
    e = pl.program_id(1)
    hf = pl.program_id(2)

    @pl.when((e == 0) & (hf == 0))
    def _():
        h = lax.dot_general(
            x_ref[...], wl_ref[...], (((1,), (1,)), ((), ())),
            precision=_PREC, preferred_element_type=jnp.float32) + bl_ref[...]
        hs_v[...] = h
        h_ref[...] = h

    tab_ref[...] = lax.dot_general(
        hs_v[...], wet_ref[0], (((1,), (1,)), ((), ())),
        precision=_PREC,
        preferred_element_type=jnp.float32) + _etype_bias(bet_ref, e, hf)


def _gruproj_body(h_ref, alo_ref, ahi_ref, wih_ref, whh_ref, bih_ref,
                  bhh_ref, wet_ref, bet_ref, tab_ref, hn_ref, hs_v):
    e = pl.program_id(1)
    hf = pl.program_id(2)

    @pl.when((e == 0) & (hf == 0))
    def _():
        a = jnp.concatenate([alo_ref[...], ahi_ref[...]], axis=1)
        hn = _gru_math(h_ref[...], a, wih_ref, whh_ref, bih_ref, bhh_ref)
        hs_v[...] = hn
        hn_ref[...] = hn

    tab_ref[...] = lax.dot_general(
        hs_v[...], wet_ref[0], (((1,), (1,)), ((), ())),
        precision=_PREC,
        preferred_element_type=jnp.float32) + _etype_bias(bet_ref, e, hf)


def _grupool_body(h_ref, alo_ref, ahi_ref, wih_ref, whh_ref, bih_ref,
                  bhh_ref, gw_ref, cw_ref, cb_ref, o_ref, m_s, s_s, v_s):
    i = pl.program_id(0)
    a = jnp.concatenate([alo_ref[...], ahi_ref[...]], axis=1)
    h = _gru_math(h_ref[...], a, wih_ref, whh_ref, bih_ref, bhh_ref)
    g = lax.dot_general(h, gw_ref[...], (((1,), (1,)), ((), ())),
                        precision=_PREC, preferred_element_type=jnp.float32)
    first = i == 0
    m_old = jnp.where(first, -jnp.inf, m_s[0, 0])
    s_old = jnp.where(first, 0.0, s_s[0, 0])
    v_old = jnp.where(first, jnp.zeros((1, _D), jnp.float32), v_s[...])
    bm = jnp.max(g)
    m_new = jnp.maximum(m_old, bm)
    corr = jnp.exp(m_old - m_new)
    p = jnp.exp(g - m_new)
    s_new = s_old * corr + jnp.sum(p)
    pv = lax.dot_general(p, h, (((0,), (0,)), ((), ())),
                         precision=_PREC, preferred_element_type=jnp.float32)
    v_new = v_old * corr + pv
    m_s[0, 0] = m_new
    s_s[0, 0] = s_new
    v_s[...] = v_new

    @pl.when(i == _NB - 1)
    def _():
        readout = v_new / s_new
        o_ref[...] = lax.dot_general(
            readout, cw_ref[...], (((1,), (1,)), ((), ())),
            precision=_PREC, preferred_element_type=jnp.float32) + cb_ref[...]


def _sc_agg_body(tab_hbm, comb_hbm, zeros_hbm, out0_hbm, out1_hbm,
                 comb_v, g0_v, g1_v, g2_v, g3_v, d_v, buf0, buf1, buf2, buf3,
                 acc, gs0, gs1, gs2, gs3):
    c = lax.axis_index("c")
    s = lax.axis_index("s")
    bufs = (buf0, buf1, buf2, buf3)
    gvs = (g0_v, g1_v, g2_v, g3_v)
    gss = (gs0, gs1, gs2, gs3)
    pltpu.sync_copy(zeros_hbm, acc.at[pl.ds(s * _RPS, _RPS)])
    pltpu.sync_copy(comb_hbm.at[pl.ds(s * _EPW, _EPW)], comb_v)
    toff = c * (_NET * _N)

    def unpack_gather_idx(j, g_ref):
        @pl.loop(0, _CH, step=16)
        def _(k):
            g_ref[pl.ds(k, 16)] = (comb_v[pl.ds(j * _CH + k, 16)]
                                   & 0x1FFFF) + toff

    def unpack_dst_idx(j):
        @pl.loop(0, _CH, step=16)
        def _(k):
            d_v[pl.ds(k, 16)] = lax.shift_right_logical(
                comb_v[pl.ds(j * _CH + k, 16)], 17)

    plsc.subcore_barrier()

    unpack_gather_idx(0, g0_v)
    pltpu.async_copy(tab_hbm.at[g0_v], buf0, gs0)
    unpack_gather_idx(1, g1_v)
    pltpu.async_copy(tab_hbm.at[g1_v], buf1, gs1)
    unpack_gather_idx(2, g2_v)
    pltpu.async_copy(tab_hbm.at[g2_v], buf2, gs2)

    @pl.loop(0, _NCH, step=4)
    def _(j):
        for b in range(4):
            jj = j + b
            b2 = (b + 3) % 4

            @pl.when(jj + 3 < _NCH)
            def _():
                unpack_gather_idx(jj + 3, gvs[b2])
                pltpu.async_copy(tab_hbm.at[gvs[b2]], bufs[b2], gss[b2])

            pltpu.make_async_copy(tab_hbm.at[gvs[b]], bufs[b], gss[b]).wait()
            unpack_dst_idx(jj)
            pltpu.sync_copy(bufs[b], acc.at[d_v], add=True)

    plsc.subcore_barrier()

    @pl.when(c == 0)
    def _():
        pltpu.sync_copy(acc.at[pl.ds(s * _RPS, _RPS)],
                        out0_hbm.at[pl.ds(s * _RPS, _RPS)])

    @pl.when(c == 1)
    def _():
        pltpu.sync_copy(acc.at[pl.ds(s * _RPS, _RPS)],
                        out1_hbm.at[pl.ds(s * _RPS, _RPS)])


def _sc_agg(tab, comb, zeros):
    f = pl.kernel(
        _sc_agg_body,
        out_type=(jax.ShapeDtypeStruct((_NPAD, 128), jnp.float32),
                  jax.ShapeDtypeStruct((_NPAD, 128), jnp.float32)),
        mesh=plsc.VectorSubcoreMesh(core_axis_name="c", subcore_axis_name="s"),
        scratch_types=[
            pltpu.VMEM((_EPW,), jnp.int32),
            pltpu.VMEM((_CH,), jnp.int32),
            pltpu.VMEM((_CH,), jnp.int32),
            pltpu.VMEM((_CH,), jnp.int32),
            pltpu.VMEM((_CH,), jnp.int32),
            pltpu.VMEM((_CH,), jnp.int32),
            pltpu.VMEM((_CH, 128), jnp.float32),
            pltpu.VMEM((_CH, 128), jnp.float32),
            pltpu.VMEM((_CH, 128), jnp.float32),
            pltpu.VMEM((_CH, 128), jnp.float32),
            pltpu.VMEM_SHARED((_NPAD, 128), jnp.float32),
            pltpu.SemaphoreType.DMA,
            pltpu.SemaphoreType.DMA,
            pltpu.SemaphoreType.DMA,
            pltpu.SemaphoreType.DMA,
        ],
    )
    return f(tab, comb, zeros)


def kernel(x, edge_index, etype, W_lin, b_lin, W_et, b_et, gru_W_ih, gru_W_hh,
           gru_b_ih, gru_b_hh, gate_W, gate_b, cls_W, cls_b):
    E = edge_index.shape[1]
    pad = _EPAD - E
    comb = (edge_index[1].astype(jnp.int32) << 17) | (
        etype.astype(jnp.int32) * _N + edge_index[0].astype(jnp.int32))
    comb = jnp.concatenate([comb, jnp.full((pad,), _N << 17, jnp.int32)])
    zeros = jnp.zeros((_RPS, 128), jnp.float32)
    b_lin2 = b_lin.reshape(1, _D)
    b_et2 = b_et.reshape(2 * _NET, 128)
    bih2 = gru_b_ih.reshape(1, 3 * _D)
    bhh2 = gru_b_hh.reshape(1, 3 * _D)
    cb2 = cls_b.reshape(1, 2)
    del gate_b

    grid3 = (_NB, _NET, 2)
    tab_spec = pl.BlockSpec(
        (_BN, 128), lambda i, e, hf: (hf * (_NET * _NB) + e * _NB + i, 0))
    wet_spec = pl.BlockSpec((1, 128, _D), lambda i, e, hf: (e, hf, 0))
    bet_spec = pl.BlockSpec((2 * _NET, 128), lambda i, e, hf: (0, 0))
    hrow3 = pl.BlockSpec((_BN, _D), lambda i, e, hf: (i, 0))
    full3 = lambda shape: pl.BlockSpec(shape, lambda i, e, hf: tuple(
        0 for _ in shape))

    tab, h = pl.pallas_call(
        _initproj_body,
        grid=grid3,
        in_specs=[hrow3, full3((_D, _D)), full3((1, _D)), wet_spec, bet_spec],
        out_specs=(tab_spec, hrow3),
        out_shape=(jax.ShapeDtypeStruct((_TROWS, 128), jnp.float32),
                   jax.ShapeDtypeStruct((_N, _D), jnp.float32)),
        scratch_shapes=[pltpu.VMEM((_BN, _D), jnp.float32)],
    )(x, W_lin, b_lin2, W_et, b_et2)

    for step in range(_STEPS):
        a_lo, a_hi = _sc_agg(tab, comb, zeros)
        if step < _STEPS - 1:
            tab, h = pl.pallas_call(
                _gruproj_body,
                grid=grid3,
                in_specs=[
                    hrow3,
                    pl.BlockSpec((_BN, 128), lambda i, e, hf: (i, 0)),
                    pl.BlockSpec((_BN, 128), lambda i, e, hf: (i, 0)),
                    full3((3 * _D, _D)), full3((3 * _D, _D)),
                    full3((1, 3 * _D)), full3((1, 3 * _D)),
                    wet_spec, bet_spec,
                ],
                out_specs=(tab_spec, hrow3),
                out_shape=(jax.ShapeDtypeStruct((_TROWS, 128), jnp.float32),
                           jax.ShapeDtypeStruct((_N, _D), jnp.float32)),
                scratch_shapes=[pltpu.VMEM((_BN, _D), jnp.float32)],
            )(h, a_lo, a_hi, gru_W_ih, gru_W_hh, bih2, bhh2, W_et, b_et2)
        else:
            result = pl.pallas_call(
                _grupool_body,
                grid=(_NB,),
                in_specs=[
                    pl.BlockSpec((_BN, _D), lambda i: (i, 0)),
                    pl.BlockSpec((_BN, 128), lambda i: (i, 0)),
                    pl.BlockSpec((_BN, 128), lambda i: (i, 0)),
                    pl.BlockSpec((3 * _D, _D), lambda i: (0, 0)),
                    pl.BlockSpec((3 * _D, _D), lambda i: (0, 0)),
                    pl.BlockSpec((1, 3 * _D), lambda i: (0, 0)),
                    pl.BlockSpec((1, 3 * _D), lambda i: (0, 0)),
                    pl.BlockSpec((1, _D), lambda i: (0, 0)),
                    pl.BlockSpec((2, _D), lambda i: (0, 0)),
                    pl.BlockSpec((1, 2), lambda i: (0, 0)),
                ],
                out_specs=pl.BlockSpec((1, 2), lambda i: (0, 0)),
                out_shape=jax.ShapeDtypeStruct((1, 2), jnp.float32),
                scratch_shapes=[
                    pltpu.SMEM((1, 1), jnp.float32),
                    pltpu.SMEM((1, 1), jnp.float32),
                    pltpu.VMEM((1, _D), jnp.float32),
                ],
            )(h, a_lo, a_hi, gru_W_ih, gru_W_hh, bih2, bhh2, gate_W, cls_W,
              cb2)
    return result


# speedup vs baseline: 1.6292x; 1.6292x over previous
import jax
import jax.numpy as jnp
from jax import lax
from jax.experimental import pallas as pl
from jax.experimental.pallas import tpu as pltpu
from jax.experimental.pallas import tpu_sc as plsc

_N = 10000
_D = 256
_NET = 4
_STEPS = 4
_BN = 2000
_NB = _N // _BN
_NPAD = 10112
_CH = 96
_NCH = 105
_EPW = _NCH * _CH
_NSUB = 16
_EPAD = _NSUB * _EPW
_TROWS = 2 * _NET * _N
_RPS = _NPAD // _NSUB
_PREC = lax.Precision.DEFAULT


def _init_body(x_ref, w_ref, b_ref, o_ref):
    o_ref[...] = lax.dot_general(
        x_ref[...], w_ref[...], (((1,), (1,)), ((), ())),
        precision=_PREC, preferred_element_type=jnp.float32) + b_ref[...]


def _proj_body(h_ref, w_ref, b_ref, o_ref):
    e = pl.program_id(1)
    hf = pl.program_id(2)
    b8 = b_ref[...]
    rowid = lax.broadcasted_iota(jnp.int32, (2 * _NET, 128), 0)
    b = jnp.sum(jnp.where(rowid == e * 2 + hf, b8, 0.0), axis=0, keepdims=True)
    o_ref[...] = lax.dot_general(
        h_ref[...], w_ref[0], (((1,), (1,)), ((), ())),
        precision=_PREC, preferred_element_type=jnp.float32) + b


def _gru_body(h_ref, alo_ref, ahi_ref, wih_ref, whh_ref, bih_ref, bhh_ref,
              o_ref):
    h = h_ref[...]
    a = jnp.concatenate([alo_ref[...], ahi_ref[...]], axis=1)
    gi = lax.dot_general(a, wih_ref[...], (((1,), (1,)), ((), ())),
                         precision=_PREC,
                         preferred_element_type=jnp.float32) + bih_ref[...]
    gh = lax.dot_general(h, whh_ref[...], (((1,), (1,)), ((), ())),
                         precision=_PREC,
                         preferred_element_type=jnp.float32) + bhh_ref[...]
    r = jax.nn.sigmoid(gi[:, :_D] + gh[:, :_D])
    z = jax.nn.sigmoid(gi[:, _D:2 * _D] + gh[:, _D:2 * _D])
    n = jnp.tanh(gi[:, 2 * _D:] + r * gh[:, 2 * _D:])
    o_ref[...] = (1.0 - z) * n + z * h


def _pool_body(h_ref, gw_ref, cw_ref, cb_ref, o_ref, m_s, s_s, v_s):
    i = pl.program_id(0)
    h = h_ref[...]
    g = lax.dot_general(h, gw_ref[...], (((1,), (1,)), ((), ())),
                        precision=_PREC, preferred_element_type=jnp.float32)
    first = i == 0
    m_old = jnp.where(first, -jnp.inf, m_s[0, 0])
    s_old = jnp.where(first, 0.0, s_s[0, 0])
    v_old = jnp.where(first, jnp.zeros((1, _D), jnp.float32), v_s[...])
    bm = jnp.max(g)
    m_new = jnp.maximum(m_old, bm)
    corr = jnp.exp(m_old - m_new)
    p = jnp.exp(g - m_new)
    s_new = s_old * corr + jnp.sum(p)
    pv = lax.dot_general(p, h, (((0,), (0,)), ((), ())),
                         precision=_PREC,
                         preferred_element_type=jnp.float32)
    v_new = v_old * corr + pv
    m_s[0, 0] = m_new
    s_s[0, 0] = s_new
    v_s[...] = v_new

    @pl.when(i == _NB - 1)
    def _():
        readout = v_new / s_new
        o_ref[...] = lax.dot_general(
            readout, cw_ref[...], (((1,), (1,)), ((), ())),
            precision=_PREC, preferred_element_type=jnp.float32) + cb_ref[...]


def _etype_bias(b_ref, e, hf):
    rowid = lax.broadcasted_iota(jnp.int32, (2 * _NET, 128), 0)
    return jnp.sum(jnp.where(rowid == e * 2 + hf, b_ref[...], 0.0), axis=0,
                   keepdims=True)


def _gru_math(h, a, wih_ref, whh_ref, bih_ref, bhh_ref):
    gi = lax.dot_general(a, wih_ref[...], (((1,), (1,)), ((), ())),
                         precision=_PREC,
                         preferred_element_type=jnp.float32) + bih_ref[...]
    gh = lax.dot_general(h, whh_ref[...], (((1,), (1,)), ((), ())),
                         precision=_PREC,
                         preferred_element_type=jnp.float32) + bhh_ref[...]
    r = jax.nn.sigmoid(gi[:, :_D] + gh[:, :_D])
    z = jax.nn.sigmoid(gi[:, _D:2 * _D] + gh[:, _D:2 * _D])
    n = jnp.tanh(gi[:, 2 * _D:] + r * gh[:, 2 * _D:])
    return (1.0 - z) * n + z * h


def _initproj_body(x_ref, wl_ref, bl_ref, wet_ref, bet_ref, tab_ref, h_ref,
                   hs_v):
    e = pl.program_id(1)
    hf = pl.program_id(2)

    @pl.when((e == 0) & (hf == 0))
    def _():
        h = lax.dot_general(
            x_ref[...], wl_ref[...], (((1,), (1,)), ((), ())),
            precision=_PREC, preferred_element_type=jnp.float32) + bl_ref[...]
        hs_v[...] = h
        h_ref[...] = h

    tab_ref[...] = lax.dot_general(
        hs_v[...], wet_ref[0], (((1,), (1,)), ((), ())),
        precision=_PREC,
        preferred_element_type=jnp.float32) + _etype_bias(bet_ref, e, hf)


def _gruproj_body(h_ref, alo_ref, ahi_ref, wih_ref, whh_ref, bih_ref,
                  bhh_ref, wet_ref, bet_ref, tab_ref, hn_ref, hs_v):
    e = pl.program_id(1)
    hf = pl.program_id(2)

    @pl.when((e == 0) & (hf == 0))
    def _():
        a = jnp.concatenate([alo_ref[...], ahi_ref[...]], axis=1)
        hn = _gru_math(h_ref[...], a, wih_ref, whh_ref, bih_ref, bhh_ref)
        hs_v[...] = hn
        hn_ref[...] = hn

    tab_ref[...] = lax.dot_general(
        hs_v[...], wet_ref[0], (((1,), (1,)), ((), ())),
        precision=_PREC,
        preferred_element_type=jnp.float32) + _etype_bias(bet_ref, e, hf)


def _grupool_body(h_ref, alo_ref, ahi_ref, wih_ref, whh_ref, bih_ref,
                  bhh_ref, gw_ref, cw_ref, cb_ref, o_ref, m_s, s_s, v_s):
    i = pl.program_id(0)
    a = jnp.concatenate([alo_ref[...], ahi_ref[...]], axis=1)
    h = _gru_math(h_ref[...], a, wih_ref, whh_ref, bih_ref, bhh_ref)
    g = lax.dot_general(h, gw_ref[...], (((1,), (1,)), ((), ())),
                        precision=_PREC, preferred_element_type=jnp.float32)
    first = i == 0
    m_old = jnp.where(first, -jnp.inf, m_s[0, 0])
    s_old = jnp.where(first, 0.0, s_s[0, 0])
    v_old = jnp.where(first, jnp.zeros((1, _D), jnp.float32), v_s[...])
    bm = jnp.max(g)
    m_new = jnp.maximum(m_old, bm)
    corr = jnp.exp(m_old - m_new)
    p = jnp.exp(g - m_new)
    s_new = s_old * corr + jnp.sum(p)
    pv = lax.dot_general(p, h, (((0,), (0,)), ((), ())),
                         precision=_PREC, preferred_element_type=jnp.float32)
    v_new = v_old * corr + pv
    m_s[0, 0] = m_new
    s_s[0, 0] = s_new
    v_s[...] = v_new

    @pl.when(i == _NB - 1)
    def _():
        readout = v_new / s_new
        o_ref[...] = lax.dot_general(
            readout, cw_ref[...], (((1,), (1,)), ((), ())),
            precision=_PREC, preferred_element_type=jnp.float32) + cb_ref[...]


def _sc_agg_body(tab_hbm, comb_hbm, zeros_hbm, out0_hbm, out1_hbm,
                 comb_v, g0_v, g1_v, g2_v, d_v, buf0, buf1, buf2, acc,
                 gs0, gs1, gs2):
    c = lax.axis_index("c")
    s = lax.axis_index("s")
    bufs = (buf0, buf1, buf2)
    gvs = (g0_v, g1_v, g2_v)
    gss = (gs0, gs1, gs2)
    pltpu.sync_copy(zeros_hbm, acc.at[pl.ds(s * _RPS, _RPS)])
    pltpu.sync_copy(comb_hbm.at[pl.ds(s * _EPW, _EPW)], comb_v)
    toff = c * (_NET * _N)

    def unpack_gather_idx(j, g_ref):
        @pl.loop(0, _CH, step=16)
        def _(k):
            g_ref[pl.ds(k, 16)] = (comb_v[pl.ds(j * _CH + k, 16)]
                                   & 0x1FFFF) + toff

    def unpack_dst_idx(j):
        @pl.loop(0, _CH, step=16)
        def _(k):
            d_v[pl.ds(k, 16)] = lax.shift_right_logical(
                comb_v[pl.ds(j * _CH + k, 16)], 17)

    plsc.subcore_barrier()

    unpack_gather_idx(0, g0_v)
    pltpu.async_copy(tab_hbm.at[g0_v], buf0, gs0)
    unpack_gather_idx(1, g1_v)
    pltpu.async_copy(tab_hbm.at[g1_v], buf1, gs1)

    @pl.loop(0, _NCH, step=3)
    def _(j):
        for b in range(3):
            jj = j + b
            b2 = (b + 2) % 3

            @pl.when(jj + 2 < _NCH)
            def _():
                unpack_gather_idx(jj + 2, gvs[b2])
                pltpu.async_copy(tab_hbm.at[gvs[b2]], bufs[b2], gss[b2])

            pltpu.make_async_copy(tab_hbm.at[gvs[b]], bufs[b], gss[b]).wait()
            unpack_dst_idx(jj)
            pltpu.sync_copy(bufs[b], acc.at[d_v], add=True)

    plsc.subcore_barrier()

    @pl.when(c == 0)
    def _():
        pltpu.sync_copy(acc.at[pl.ds(s * _RPS, _RPS)],
                        out0_hbm.at[pl.ds(s * _RPS, _RPS)])

    @pl.when(c == 1)
    def _():
        pltpu.sync_copy(acc.at[pl.ds(s * _RPS, _RPS)],
                        out1_hbm.at[pl.ds(s * _RPS, _RPS)])


def _sc_agg(tab, comb, zeros):
    f = pl.kernel(
        _sc_agg_body,
        out_type=(jax.ShapeDtypeStruct((_NPAD, 128), jnp.float32),
                  jax.ShapeDtypeStruct((_NPAD, 128), jnp.float32)),
        mesh=plsc.VectorSubcoreMesh(core_axis_name="c", subcore_axis_name="s"),
        scratch_types=[
            pltpu.VMEM((_EPW,), jnp.int32),
            pltpu.VMEM((_CH,), jnp.int32),
            pltpu.VMEM((_CH,), jnp.int32),
            pltpu.VMEM((_CH,), jnp.int32),
            pltpu.VMEM((_CH,), jnp.int32),
            pltpu.VMEM((_CH, 128), jnp.float32),
            pltpu.VMEM((_CH, 128), jnp.float32),
            pltpu.VMEM((_CH, 128), jnp.float32),
            pltpu.VMEM_SHARED((_NPAD, 128), jnp.float32),
            pltpu.SemaphoreType.DMA,
            pltpu.SemaphoreType.DMA,
            pltpu.SemaphoreType.DMA,
        ],
    )
    return f(tab, comb, zeros)


def kernel(x, edge_index, etype, W_lin, b_lin, W_et, b_et, gru_W_ih, gru_W_hh,
           gru_b_ih, gru_b_hh, gate_W, gate_b, cls_W, cls_b):
    E = edge_index.shape[1]
    pad = _EPAD - E
    comb = (edge_index[1].astype(jnp.int32) << 17) | (
        etype.astype(jnp.int32) * _N + edge_index[0].astype(jnp.int32))
    comb = jnp.concatenate([comb, jnp.full((pad,), _N << 17, jnp.int32)])
    zeros = jnp.zeros((_RPS, 128), jnp.float32)
    b_lin2 = b_lin.reshape(1, _D)
    b_et2 = b_et.reshape(2 * _NET, 128)
    bih2 = gru_b_ih.reshape(1, 3 * _D)
    bhh2 = gru_b_hh.reshape(1, 3 * _D)
    cb2 = cls_b.reshape(1, 2)
    del gate_b

    grid3 = (_NB, _NET, 2)
    tab_spec = pl.BlockSpec(
        (_BN, 128), lambda i, e, hf: (hf * (_NET * _NB) + e * _NB + i, 0))
    wet_spec = pl.BlockSpec((1, 128, _D), lambda i, e, hf: (e, hf, 0))
    bet_spec = pl.BlockSpec((2 * _NET, 128), lambda i, e, hf: (0, 0))
    hrow3 = pl.BlockSpec((_BN, _D), lambda i, e, hf: (i, 0))
    full3 = lambda shape: pl.BlockSpec(shape, lambda i, e, hf: tuple(
        0 for _ in shape))

    tab, h = pl.pallas_call(
        _initproj_body,
        grid=grid3,
        in_specs=[hrow3, full3((_D, _D)), full3((1, _D)), wet_spec, bet_spec],
        out_specs=(tab_spec, hrow3),
        out_shape=(jax.ShapeDtypeStruct((_TROWS, 128), jnp.float32),
                   jax.ShapeDtypeStruct((_N, _D), jnp.float32)),
        scratch_shapes=[pltpu.VMEM((_BN, _D), jnp.float32)],
    )(x, W_lin, b_lin2, W_et, b_et2)

    for step in range(_STEPS):
        a_lo, a_hi = _sc_agg(tab, comb, zeros)
        if step < _STEPS - 1:
            tab, h = pl.pallas_call(
                _gruproj_body,
                grid=grid3,
                in_specs=[
                    hrow3,
                    pl.BlockSpec((_BN, 128), lambda i, e, hf: (i, 0)),
                    pl.BlockSpec((_BN, 128), lambda i, e, hf: (i, 0)),
                    full3((3 * _D, _D)), full3((3 * _D, _D)),
                    full3((1, 3 * _D)), full3((1, 3 * _D)),
                    wet_spec, bet_spec,
                ],
                out_specs=(tab_spec, hrow3),
                out_shape=(jax.ShapeDtypeStruct((_TROWS, 128), jnp.float32),
                           jax.ShapeDtypeStruct((_N, _D), jnp.float32)),
                scratch_shapes=[pltpu.VMEM((_BN, _D), jnp.float32)],
            )(h, a_lo, a_hi, gru_W_ih, gru_W_hh, bih2, bhh2, W_et, b_et2)
        else:
            result = pl.pallas_call(
                _grupool_body,
                grid=(_NB,),
                in_specs=[
                    pl.BlockSpec((_BN, _D), lambda i: (i, 0)),
                    pl.BlockSpec((_BN, 128), lambda i: (i, 0)),
                    pl.BlockSpec((_BN, 128), lambda i: (i, 0)),
                    pl.BlockSpec((3 * _D, _D), lambda i: (0, 0)),
                    pl.BlockSpec((3 * _D, _D), lambda i: (0, 0)),
                    pl.BlockSpec((1, 3 * _D), lambda i: (0, 0)),
                    pl.BlockSpec((1, 3 * _D), lambda i: (0, 0)),
                    pl.BlockSpec((1, _D), lambda i: (0, 0)),
                    pl.BlockSpec((2, _D), lambda i: (0, 0)),
                    pl.BlockSpec((1, 2), lambda i: (0, 0)),
                ],
                out_specs=pl.BlockSpec((1, 2), lambda i: (0, 0)),
                out_shape=jax.ShapeDtypeStruct((1, 2), jnp.float32),
                scratch_shapes=[
                    pltpu.SMEM((1, 1), jnp.float32),
                    pltpu.SMEM((1, 1), jnp.float32),
                    pltpu.VMEM((1, _D), jnp.float32),
                ],
            )(h, a_lo, a_hi, gru_W_ih, gru_W_hh, bih2, bhh2, gate_W, cls_W,
              cb2)
    return result
